# baseline (device time: 36628 ns/iter reference)
import jax
import jax.numpy as jnp
from jax import lax
from jax.experimental import pallas as pl
from jax.experimental.pallas import tpu as pltpu

N_DEV = 32
B, SQ, D = 2, 128, 512
HQ_LOC, HKV_LOC, DH = 8, 2, 64
ROWS = B * SQ
CH = ROWS // N_DEV


def _fused(xb, wqb, wob, kb, vb):
    bf = jnp.bfloat16

    def body(x_ref, wq_ref, wo_ref, k_ref, v_ref, out_ref, p_scr,
             sb0, sb1, sb2, sb3, sb4, rb0, rb1, rb2, rb3, rb4,
             send_sem, rs_sems, ag_sems):
        send_bufs = [sb0, sb1, sb2, sb3, sb4]
        rs_bufs = [rb0, rb1, rb2, rb3, rb4]

        my = lax.axis_index("i")
        z = my // 8
        p = my % 8
        y = p // 2
        x = (p % 2) ^ (y & 1)

        def ridx(xx, yy, zz):
            return zz * 8 + 2 * yy + (xx ^ (yy & 1))

        partners = [
            ridx(x, y ^ 1, z),
            ridx(x ^ 1, y, z),
            ridx(x, y, z ^ 1),
            ridx(x, y ^ 2, z),
            ridx(x, y, z ^ 2),
        ]
        tbits = [y & 1, x, z & 1, (y >> 1) & 1, (z >> 1) & 1]

        barrier = pltpu.get_barrier_semaphore()
        for pr in partners:
            pl.semaphore_signal(barrier, inc=1, device_id=(pr,),
                                device_id_type=pl.DeviceIdType.MESH)

        for b in range(B):
            q = jnp.dot(x_ref[b], wq_ref[...],
                        preferred_element_type=jnp.float32)
            q = q.astype(bf)
            o_cols = []
            for kv in range(HKV_LOC):
                kk = k_ref[b, :, kv, :]
                vv = v_ref[b, :, kv, :]
                for hh in range(4):
                    c0 = (kv * 4 + hh) * DH
                    qh = q[:, c0:c0 + DH]
                    s = lax.dot_general(
                        qh, kk, (((1,), (1,)), ((), ())),
                        preferred_element_type=jnp.float32) * 0.125
                    m = jnp.max(s, axis=1, keepdims=True)
                    e = jnp.exp(s - m)
                    l = jnp.sum(e, axis=1, keepdims=True)
                    o_cols.append(jnp.dot((e / l).astype(bf), vv,
                                          preferred_element_type=jnp.float32))
            o_b = jnp.concatenate(o_cols, axis=1).astype(bf)
            p_scr[pl.ds(b * SQ, SQ), :] = jnp.dot(
                o_b, wo_ref[...], preferred_element_type=jnp.float32)

        pl.semaphore_wait(barrier, 5)

        lo = 0
        for s in range(5):
            half = 128 >> s
            keep_lo = lo + tbits[s] * half
            send_lo = lo + (1 - tbits[s]) * half
            send_bufs[s][...] = p_scr[pl.ds(send_lo, half), :].astype(bf)
            rdma = pltpu.make_async_remote_copy(
                src_ref=send_bufs[s],
                dst_ref=rs_bufs[s],
                send_sem=send_sem,
                recv_sem=rs_sems.at[s],
                device_id=(partners[s],),
                device_id_type=pl.DeviceIdType.MESH,
            )
            rdma.start()
            rdma.wait()
            recv = rs_bufs[s][...].astype(jnp.float32)
            if s < 4:
                p_scr[pl.ds(keep_lo, half), :] = (
                    p_scr[pl.ds(keep_lo, half), :] + recv)
            else:
                out_ref[pl.ds(keep_lo, half), :] = (
                    p_scr[pl.ds(keep_lo, half), :] + recv).astype(bf)
            lo = keep_lo

        for s in reversed(range(5)):
            glen = 128 >> s
            rdma = pltpu.make_async_remote_copy(
                src_ref=out_ref.at[pl.ds(lo, glen)],
                dst_ref=out_ref.at[pl.ds(lo, glen)],
                send_sem=send_sem,
                recv_sem=ag_sems.at[s],
                device_id=(partners[s],),
                device_id_type=pl.DeviceIdType.MESH,
            )
            rdma.start()
            rdma.wait()
            lo = lo - tbits[s] * glen

    return pl.pallas_call(
        body,
        out_shape=jax.ShapeDtypeStruct((ROWS, D), jnp.bfloat16),
        in_specs=[pl.BlockSpec(memory_space=pltpu.VMEM)] * 5,
        out_specs=pl.BlockSpec(memory_space=pltpu.VMEM),
        scratch_shapes=[
            pltpu.VMEM((ROWS, D), jnp.float32),
            pltpu.VMEM((128, D), jnp.bfloat16),
            pltpu.VMEM((64, D), jnp.bfloat16),
            pltpu.VMEM((32, D), jnp.bfloat16),
            pltpu.VMEM((16, D), jnp.bfloat16),
            pltpu.VMEM((8, D), jnp.bfloat16),
            pltpu.VMEM((128, D), jnp.bfloat16),
            pltpu.VMEM((64, D), jnp.bfloat16),
            pltpu.VMEM((32, D), jnp.bfloat16),
            pltpu.VMEM((16, D), jnp.bfloat16),
            pltpu.VMEM((8, D), jnp.bfloat16),
            pltpu.SemaphoreType.DMA,
            pltpu.SemaphoreType.DMA((5,)),
            pltpu.SemaphoreType.DMA((5,)),
        ],
        compiler_params=pltpu.CompilerParams(collective_id=0),
    )(xb, wqb, wob, kb, vb)


def kernel(x, Wq, Wo, K_ext, V_ext):
    my = lax.axis_index("i")
    bf = jnp.bfloat16

    k_loc = lax.dynamic_slice_in_dim(K_ext, my * HKV_LOC, HKV_LOC, axis=2)
    v_loc = lax.dynamic_slice_in_dim(V_ext, my * HKV_LOC, HKV_LOC, axis=2)

    out = _fused(x.astype(bf), Wq.astype(bf), Wo.astype(bf),
                 k_loc.astype(bf), v_loc.astype(bf))
    return out.reshape(B, SQ, D)


# device time: 34327 ns/iter; 1.0670x vs baseline; 1.0670x over previous
import jax
import jax.numpy as jnp
from jax import lax
from jax.experimental import pallas as pl
from jax.experimental.pallas import tpu as pltpu

N_DEV = 32
B, SQ, D = 2, 128, 512
HQ_LOC, HKV_LOC, DH = 8, 2, 64
ROWS = B * SQ
CH = ROWS // N_DEV


def _fused(xb, wqb, wob, kb, vb):
    bf = jnp.bfloat16

    def body(x_ref, wq_ref, wo_ref, k_ref, v_ref, out_ref, p_scr,
             sb0, sb1, sb2, sb3, sb4, rb0, rb1, rb2, rb3, rb4,
             send_sem, rs_sems, ag_sems):
        send_bufs = [sb0, sb1, sb2, sb3, sb4]
        rs_bufs = [rb0, rb1, rb2, rb3, rb4]

        my = lax.axis_index("i")
        z = my // 8
        p = my % 8
        y = p // 2
        x = (p % 2) ^ (y & 1)

        def ridx(xx, yy, zz):
            return zz * 8 + 2 * yy + (xx ^ (yy & 1))

        partners = [
            ridx(x, y ^ 1, z),
            ridx(x ^ 1, y, z),
            ridx(x, y, z ^ 1),
            ridx(x, y ^ 2, z),
            ridx(x, y, z ^ 2),
        ]
        tbits = [y & 1, x, z & 1, (y >> 1) & 1, (z >> 1) & 1]

        barrier = pltpu.get_barrier_semaphore()
        for pr in partners:
            pl.semaphore_signal(barrier, inc=1, device_id=(pr,),
                                device_id_type=pl.DeviceIdType.MESH)

        wqb = wq_ref[...].astype(bf)
        wob = wo_ref[...].astype(bf)

        def compute_batch(b):
            xq = x_ref[pl.ds(b, 1)].reshape(SQ, D).astype(bf)
            kk2 = k_ref[pl.ds(b, 1)].reshape(SQ, HKV_LOC, DH)
            vv2 = v_ref[pl.ds(b, 1)].reshape(SQ, HKV_LOC, DH)
            q = jnp.dot(xq, wqb, preferred_element_type=jnp.float32)
            q = q.astype(bf)
            o_cols = []
            for kv in range(HKV_LOC):
                kk = kk2[:, kv, :].astype(bf)
                vv = vv2[:, kv, :].astype(bf)
                for hh in range(4):
                    c0 = (kv * 4 + hh) * DH
                    qh = q[:, c0:c0 + DH]
                    s = lax.dot_general(
                        qh, kk, (((1,), (1,)), ((), ())),
                        preferred_element_type=jnp.float32) * 0.125
                    m = jnp.max(s, axis=1, keepdims=True)
                    e = jnp.exp(s - m)
                    l = jnp.sum(e, axis=1, keepdims=True)
                    o_cols.append(jnp.dot((e / l).astype(bf), vv,
                                          preferred_element_type=jnp.float32))
            o_b = jnp.concatenate(o_cols, axis=1).astype(bf)
            return jnp.dot(o_b, wob, preferred_element_type=jnp.float32)

        t0 = tbits[0]
        sb = 1 - t0
        part_send = compute_batch(sb)
        p_scr[pl.ds(sb * SQ, SQ), :] = part_send
        send_bufs[0][...] = part_send.astype(bf)

        pl.semaphore_wait(barrier, 5)
        rdma0 = pltpu.make_async_remote_copy(
            src_ref=send_bufs[0],
            dst_ref=rs_bufs[0],
            send_sem=send_sem,
            recv_sem=rs_sems.at[0],
            device_id=(partners[0],),
            device_id_type=pl.DeviceIdType.MESH,
        )
        rdma0.start()
        p_scr[pl.ds(t0 * SQ, SQ), :] = compute_batch(t0)
        rdma0.wait()
        p_scr[pl.ds(t0 * SQ, SQ), :] = (
            p_scr[pl.ds(t0 * SQ, SQ), :]
            + rs_bufs[0][...].astype(jnp.float32))
        lo = t0 * SQ

        for s in range(1, 5):
            half = 128 >> s
            keep_lo = lo + tbits[s] * half
            send_lo = lo + (1 - tbits[s]) * half
            send_bufs[s][...] = p_scr[pl.ds(send_lo, half), :].astype(bf)
            rdma = pltpu.make_async_remote_copy(
                src_ref=send_bufs[s],
                dst_ref=rs_bufs[s],
                send_sem=send_sem,
                recv_sem=rs_sems.at[s],
                device_id=(partners[s],),
                device_id_type=pl.DeviceIdType.MESH,
            )
            rdma.start()
            rdma.wait()
            recv = rs_bufs[s][...].astype(jnp.float32)
            if s < 4:
                p_scr[pl.ds(keep_lo, half), :] = (
                    p_scr[pl.ds(keep_lo, half), :] + recv)
            else:
                out_ref[pl.ds(keep_lo, half), :] = (
                    p_scr[pl.ds(keep_lo, half), :] + recv).astype(bf)
            lo = keep_lo

        for s in reversed(range(5)):
            glen = 128 >> s
            rdma = pltpu.make_async_remote_copy(
                src_ref=out_ref.at[pl.ds(lo, glen)],
                dst_ref=out_ref.at[pl.ds(lo, glen)],
                send_sem=send_sem,
                recv_sem=ag_sems.at[s],
                device_id=(partners[s],),
                device_id_type=pl.DeviceIdType.MESH,
            )
            rdma.start()
            rdma.wait()
            lo = lo - tbits[s] * glen

    return pl.pallas_call(
        body,
        out_shape=jax.ShapeDtypeStruct((ROWS, D), jnp.bfloat16),
        in_specs=[pl.BlockSpec(memory_space=pltpu.VMEM)] * 5,
        out_specs=pl.BlockSpec(memory_space=pltpu.VMEM),
        scratch_shapes=[
            pltpu.VMEM((ROWS, D), jnp.float32),
            pltpu.VMEM((128, D), jnp.bfloat16),
            pltpu.VMEM((64, D), jnp.bfloat16),
            pltpu.VMEM((32, D), jnp.bfloat16),
            pltpu.VMEM((16, D), jnp.bfloat16),
            pltpu.VMEM((8, D), jnp.bfloat16),
            pltpu.VMEM((128, D), jnp.bfloat16),
            pltpu.VMEM((64, D), jnp.bfloat16),
            pltpu.VMEM((32, D), jnp.bfloat16),
            pltpu.VMEM((16, D), jnp.bfloat16),
            pltpu.VMEM((8, D), jnp.bfloat16),
            pltpu.SemaphoreType.DMA,
            pltpu.SemaphoreType.DMA((5,)),
            pltpu.SemaphoreType.DMA((5,)),
        ],
        compiler_params=pltpu.CompilerParams(collective_id=0),
    )(xb, wqb, wob, kb, vb)


def kernel(x, Wq, Wo, K_ext, V_ext):
    my = lax.axis_index("i")

    k_loc = lax.dynamic_slice_in_dim(K_ext, my * HKV_LOC, HKV_LOC, axis=2)
    v_loc = lax.dynamic_slice_in_dim(V_ext, my * HKV_LOC, HKV_LOC, axis=2)

    out = _fused(x, Wq, Wo, k_loc, v_loc)
    return out.reshape(B, SQ, D)
